# baseline (speedup 1.0000x reference)
import functools

import jax
import jax.numpy as jnp
from jax import lax
from jax.experimental import pallas as pl
from jax.experimental.pallas import tpu as pltpu
from jax.experimental.pallas import tpu_sc as plsc

VOCAB = 49408
EMBED = 1024
MAX_POS = 77
BATCH = 1024
SEQ = 77

RP = 80
NW = 32
RPW = BATCH * RP // NW
CHUNK = 16
NCHUNK = RPW // CHUNK
NBUF = 4

TCB = 8


def _sc_body(ids_hbm, tok_hbm, out_hbm,
             idx0, idx1, idx2, idx3, rows0, rows1, rows2, rows3,
             gs0, gs1, gs2, gs3, ss0, ss1, ss2, ss3):
    wid = lax.axis_index("s") * 2 + lax.axis_index("c")
    base = wid * RPW

    idxs = (idx0, idx1, idx2, idx3)
    rows = (rows0, rows1, rows2, rows3)
    gsems = (gs0, gs1, gs2, gs3)
    ssems = (ss0, ss1, ss2, ss3)

    def stage_and_gather(c, q):
        pltpu.sync_copy(ids_hbm.at[pl.ds(base + c * CHUNK, CHUNK)], idxs[q])
        pltpu.async_copy(tok_hbm.at[idxs[q]], rows[q], gsems[q])

    def wait_gather(q):
        pltpu.make_async_copy(tok_hbm.at[idxs[q]], rows[q], gsems[q]).wait()

    def start_store(c, q):
        pltpu.async_copy(rows[q],
                         out_hbm.at[pl.ds(base + c * CHUNK, CHUNK)], ssems[q])

    def wait_store(c, q):
        pltpu.make_async_copy(rows[q],
                              out_hbm.at[pl.ds(base + c * CHUNK, CHUNK)],
                              ssems[q]).wait()

    stage_and_gather(0, 0)
    stage_and_gather(1, 1)

    def quad_body(t, carry):
        for q in (0, 1, 2, 3):
            c = 4 * t + q
            wait_gather(q)
            start_store(c, q)
            qn = (q + 2) % NBUF
            @pl.when(c + 2 < NCHUNK)
            def _():
                @pl.when(c >= 2)
                def _():
                    wait_store(c - 2, qn)
                stage_and_gather(c + 2, qn)
        return carry

    lax.fori_loop(0, NCHUNK // NBUF, quad_body, 0)
    wait_store(NCHUNK - 2, (NCHUNK - 2) % NBUF)
    wait_store(NCHUNK - 1, (NCHUNK - 1) % NBUF)


def _tc_body(gat_ref, pos_ref, out_ref):
    pos = pos_ref[...]
    for k in range(TCB):
        out_ref[k] = gat_ref[pl.ds(k * RP, SEQ), :] + pos


@jax.jit
def kernel(input_ids, token_table, position_table):
    ids = jnp.pad(input_ids.astype(jnp.int32), ((0, 0), (0, RP - SEQ)))
    ids = ids.reshape(-1)
    mesh = plsc.VectorSubcoreMesh(core_axis_name="c", subcore_axis_name="s")
    gathered = pl.kernel(
        _sc_body,
        mesh=mesh,
        out_type=jax.ShapeDtypeStruct((BATCH * RP, EMBED), jnp.float32),
        scratch_types=(
            [pltpu.VMEM((CHUNK,), jnp.int32)] * NBUF
            + [pltpu.VMEM((CHUNK, EMBED), jnp.float32)] * NBUF
            + [pltpu.SemaphoreType.DMA] * (2 * NBUF)
        ),
    )(ids, token_table)

    return pl.pallas_call(
        _tc_body,
        grid=(BATCH // TCB,),
        in_specs=[
            pl.BlockSpec((TCB * RP, EMBED), lambda i: (i, 0)),
            pl.BlockSpec((SEQ, EMBED), lambda i: (0, 0)),
        ],
        out_specs=pl.BlockSpec((TCB, SEQ, EMBED), lambda i: (i, 0, 0)),
        out_shape=jax.ShapeDtypeStruct((BATCH, SEQ, EMBED), jnp.float32),
        compiler_params=pltpu.CompilerParams(
            dimension_semantics=("arbitrary",)),
    )(gathered, position_table)


# speedup vs baseline: 1.3108x; 1.3108x over previous
import functools

import jax
import jax.numpy as jnp
from jax import lax
from jax.experimental import pallas as pl
from jax.experimental.pallas import tpu as pltpu
from jax.experimental.pallas import tpu_sc as plsc

VOCAB = 49408
EMBED = 1024
MAX_POS = 77
BATCH = 1024
SEQ = 77

NQ = 4
QD = EMBED // NQ
NG = 8
B_PER_G = BATCH // NG
RP = 80
LANES = 16


def _add_pos(rows_v, pos_v, out_v):
    def row_body(r, carry):
        for j in range(QD // LANES):
            sl = pl.ds(j * LANES, LANES)
            out_v[r, sl] = rows_v[r, sl] + pos_v[r, sl]
        return carry

    lax.fori_loop(0, MAX_POS, row_body, 0)


def _body(ids_hbm, tok_hbm, pos_hbm, out_hbm,
          idx0, idx1, rows0, rows1, st0, st1, pos_v,
          gsem0, gsem1, ssem0, ssem1):
    wid = lax.axis_index("s") * 2 + lax.axis_index("c")
    g = wid // NQ
    h = wid % NQ
    b0 = g * B_PER_G
    col = h * QD

    idxs = (idx0, idx1)
    rows = (rows0, rows1)
    sts = (st0, st1)
    gsems = (gsem0, gsem1)
    ssems = (ssem0, ssem1)

    pltpu.sync_copy(pos_hbm.at[:, pl.ds(col, QD)], pos_v)

    def stage_and_gather(c, p):
        pltpu.sync_copy(ids_hbm.at[pl.ds((b0 + c) * RP, RP)], idxs[p])
        pltpu.async_copy(tok_hbm.at[idxs[p], pl.ds(col, QD)], rows[p],
                         gsems[p])

    def wait_gather(p):
        pltpu.make_async_copy(tok_hbm.at[idxs[p], pl.ds(col, QD)], rows[p],
                              gsems[p]).wait()

    def start_store(c, p):
        pltpu.async_copy(sts[p], out_hbm.at[b0 + c, :, pl.ds(col, QD)],
                         ssems[p])

    def wait_store(c, p):
        pltpu.make_async_copy(sts[p], out_hbm.at[b0 + c, :, pl.ds(col, QD)],
                              ssems[p]).wait()

    stage_and_gather(0, 0)

    def pair_body(t, carry):
        for p in (0, 1):
            c = 2 * t + p
            if p == 0:
                stage_and_gather(c + 1, 1)
            else:
                @pl.when(t < B_PER_G // 2 - 1)
                def _():
                    stage_and_gather(c + 1, 0)
            wait_gather(p)
            @pl.when(t > 0)
            def _():
                wait_store(c - 2, p)
            _add_pos(rows[p], pos_v, sts[p])
            start_store(c, p)
        return carry

    lax.fori_loop(0, B_PER_G // 2, pair_body, 0)
    wait_store(B_PER_G - 2, 0)
    wait_store(B_PER_G - 1, 1)


@jax.jit
def kernel(input_ids, token_table, position_table):
    ids = jnp.pad(input_ids.astype(jnp.int32), ((0, 0), (0, RP - SEQ)))
    ids = ids.reshape(-1)
    mesh = plsc.VectorSubcoreMesh(core_axis_name="c", subcore_axis_name="s")
    return pl.kernel(
        _body,
        mesh=mesh,
        out_type=jax.ShapeDtypeStruct((BATCH, SEQ, EMBED), jnp.float32),
        scratch_types=[
            pltpu.VMEM((RP,), jnp.int32),
            pltpu.VMEM((RP,), jnp.int32),
            pltpu.VMEM((RP, QD), jnp.float32),
            pltpu.VMEM((RP, QD), jnp.float32),
            pltpu.VMEM((MAX_POS, QD), jnp.float32),
            pltpu.VMEM((MAX_POS, QD), jnp.float32),
            pltpu.VMEM((MAX_POS, QD), jnp.float32),
            pltpu.SemaphoreType.DMA,
            pltpu.SemaphoreType.DMA,
            pltpu.SemaphoreType.DMA,
            pltpu.SemaphoreType.DMA,
        ],
    )(ids, token_table, position_table)


# speedup vs baseline: 1.3215x; 1.0081x over previous
import jax
import jax.numpy as jnp
from jax import lax
from jax.experimental import pallas as pl
from jax.experimental.pallas import tpu as pltpu
from jax.experimental.pallas import tpu_sc as plsc

VOCAB = 49408
EMBED = 1024
MAX_POS = 77
BATCH = 1024
SEQ = 77

NQ = 4
QD = EMBED // NQ
NG = 8
B_PER_G = BATCH // NG
RP = 80
LANES = 16


def _add_pos(rows_v, pos_v, out_v):
    def row_body(r, carry):
        for j in range(QD // LANES):
            sl = pl.ds(j * LANES, LANES)
            out_v[r, sl] = rows_v[r, sl] + pos_v[r, sl]
        return carry

    lax.fori_loop(0, MAX_POS, row_body, 0)


def _body(ids_hbm, tok_hbm, pos_hbm, out_hbm,
          idx_v, rows0, rows1, st0, st1, pos_v,
          gsem0, gsem1, ssem0, ssem1):
    wid = lax.axis_index("s") * 2 + lax.axis_index("c")
    g = wid // NQ
    h = wid % NQ
    b0 = g * B_PER_G
    col = h * QD

    rows = (rows0, rows1)
    sts = (st0, st1)
    gsems = (gsem0, gsem1)
    ssems = (ssem0, ssem1)

    pltpu.sync_copy(pos_hbm.at[:, pl.ds(col, QD)], pos_v)
    pltpu.sync_copy(ids_hbm.at[pl.ds(b0 * RP, B_PER_G * RP)], idx_v)

    def stage_and_gather(c, p):
        pltpu.async_copy(
            tok_hbm.at[idx_v.at[pl.ds(c * RP, RP)], pl.ds(col, QD)],
            rows[p], gsems[p])

    def wait_gather(c, p):
        pltpu.make_async_copy(
            tok_hbm.at[idx_v.at[pl.ds(c * RP, RP)], pl.ds(col, QD)],
            rows[p], gsems[p]).wait()

    def start_store(c, p):
        pltpu.async_copy(sts[p], out_hbm.at[b0 + c, :, pl.ds(col, QD)],
                         ssems[p])

    def wait_store(c, p):
        pltpu.make_async_copy(sts[p], out_hbm.at[b0 + c, :, pl.ds(col, QD)],
                              ssems[p]).wait()

    stage_and_gather(0, 0)

    def pair_body(t, carry):
        for p in (0, 1):
            c = 2 * t + p
            if p == 0:
                stage_and_gather(c + 1, 1)
            else:
                @pl.when(t < B_PER_G // 2 - 1)
                def _():
                    stage_and_gather(c + 1, 0)
            wait_gather(c, p)
            @pl.when(t > 0)
            def _():
                wait_store(c - 2, p)
            _add_pos(rows[p], pos_v, sts[p])
            start_store(c, p)
        return carry

    lax.fori_loop(0, B_PER_G // 2, pair_body, 0)
    wait_store(B_PER_G - 2, 0)
    wait_store(B_PER_G - 1, 1)


@jax.jit
def kernel(input_ids, token_table, position_table):
    ids = jnp.pad(input_ids.astype(jnp.int32), ((0, 0), (0, RP - SEQ)))
    ids = ids.reshape(-1)
    mesh = plsc.VectorSubcoreMesh(core_axis_name="c", subcore_axis_name="s")
    return pl.kernel(
        _body,
        mesh=mesh,
        out_type=jax.ShapeDtypeStruct((BATCH, SEQ, EMBED), jnp.float32),
        scratch_types=[
            pltpu.VMEM((B_PER_G * RP,), jnp.int32),
            pltpu.VMEM((RP, QD), jnp.float32),
            pltpu.VMEM((RP, QD), jnp.float32),
            pltpu.VMEM((MAX_POS, QD), jnp.float32),
            pltpu.VMEM((MAX_POS, QD), jnp.float32),
            pltpu.VMEM((MAX_POS, QD), jnp.float32),
            pltpu.SemaphoreType.DMA,
            pltpu.SemaphoreType.DMA,
            pltpu.SemaphoreType.DMA,
            pltpu.SemaphoreType.DMA,
        ],
    )(ids, token_table, position_table)
